# baseline (speedup 1.0000x reference)
import functools
import math

import jax
import jax.numpy as jnp
from jax import lax
from jax.experimental import pallas as pl
from jax.experimental.pallas import tpu as pltpu
from jax.experimental.pallas import tpu_sc as plsc

ROWS = 1024
VOCAB = 100000
N_TOTAL = ROWS * VOCAB
LABEL_SMOOTHING = 0.1
CONFIDENCE = 1.0 - LABEL_SMOOTHING
FILL = LABEL_SMOOTHING / (VOCAB - 1)
C0 = ((N_TOTAL - ROWS) * FILL * math.log(FILL)
      + ROWS * CONFIDENCE * math.log(CONFIDENCE)) / N_TOTAL

VB = 5000
GRID = VOCAB // VB


def _tc_body(x_ref, out_ref):
    i = pl.program_id(0)

    @pl.when(i == 0)
    def _init():
        out_ref[0, 0] = 0.0

    out_ref[0, 0] += jnp.sum(x_ref[...])


def _tc_call(pred_t):
    return pl.pallas_call(
        _tc_body,
        grid=(GRID,),
        in_specs=[pl.BlockSpec((VB, ROWS), lambda i: (i, 0))],
        out_specs=pl.BlockSpec(memory_space=pltpu.SMEM),
        out_shape=jax.ShapeDtypeStruct((1, 1), jnp.float32),
    )(pred_t)


NUM_CORES = 2
NUM_SUBCORES = 16
NUM_WORKERS = NUM_CORES * NUM_SUBCORES
CPW = ROWS // NUM_WORKERS
SC_LANES = 16
TILE_R = 8
TILE_C = 128


def _sc_body(pred_t_hbm, target_hbm, g_out, tgt_v, buf0, buf1, res_v,
             sem0, sem1):
    wid = lax.axis_index("s") * NUM_CORES + lax.axis_index("c")
    col0 = wid * CPW
    pltpu.sync_copy(target_hbm.at[pl.ds(col0, CPW)], tgt_v)

    iota = lax.broadcasted_iota(jnp.int32, (SC_LANES,), 0)
    tvecs = [tgt_v[pl.ds(k * SC_LANES, SC_LANES)]
             for k in range(CPW // SC_LANES)]

    bufs = (buf0, buf1)
    sems = (sem0, sem1)

    def start(j):
        ts = tvecs[j // SC_LANES][j % SC_LANES]
        cs = col0 + j
        g8 = pl.multiple_of((ts // TILE_R) * TILE_R, TILE_R)
        c128 = pl.multiple_of((cs // TILE_C) * TILE_C, TILE_C)
        return pltpu.async_copy(
            pred_t_hbm.at[pl.ds(g8, TILE_R), pl.ds(c128, TILE_C)],
            bufs[j % 2], sems[j % 2])

    acc_g = jnp.zeros((SC_LANES,), jnp.float32)
    pending = [start(0), start(1)]
    for j in range(CPW):
        pending[j % 2].wait()
        ts = tvecs[j // SC_LANES][j % SC_LANES]
        cs = col0 + j
        rsub = ts % TILE_R
        cmod = cs % TILE_C
        l16 = pl.multiple_of((cmod // SC_LANES) * SC_LANES, SC_LANES)
        wvec = bufs[j % 2][rsub, pl.ds(l16, SC_LANES)]
        acc_g += jnp.where(iota == cmod - l16, wvec, 0.0)
        if j + 2 < CPW:
            pending[j % 2] = start(j + 2)

    res_v[...] = acc_g
    pltpu.sync_copy(res_v, g_out.at[wid])


_sc_call = functools.partial(
    pl.kernel,
    mesh=plsc.VectorSubcoreMesh(core_axis_name="c", subcore_axis_name="s"),
    out_type=jax.ShapeDtypeStruct((NUM_WORKERS, SC_LANES), jnp.float32),
    scratch_types=[
        pltpu.VMEM((CPW,), jnp.int32),
        pltpu.VMEM((TILE_R, TILE_C), jnp.float32),
        pltpu.VMEM((TILE_R, TILE_C), jnp.float32),
        pltpu.VMEM((SC_LANES,), jnp.float32),
        pltpu.SemaphoreType.DMA,
        pltpu.SemaphoreType.DMA,
    ],
)(_sc_body)


def kernel(pred, target):
    pred_t = pred.T
    g_partials = _sc_call(pred_t, target)
    s_out = _tc_call(pred_t)
    s_total = s_out[0, 0]
    g_total = jnp.sum(g_partials)
    loss = (jnp.float32(C0)
            - (jnp.float32(FILL) * s_total
               + jnp.float32(CONFIDENCE - FILL) * g_total)
            * jnp.float32(1.0 / N_TOTAL))
    return loss


# speedup vs baseline: 1.1444x; 1.1444x over previous
import math

import jax
import jax.numpy as jnp
from jax import lax
from jax.experimental import pallas as pl
from jax.experimental.pallas import tpu as pltpu

ROWS = 1024
VOCAB = 100000
N_TOTAL = ROWS * VOCAB
LABEL_SMOOTHING = 0.1
CONFIDENCE = 1.0 - LABEL_SMOOTHING
FILL = LABEL_SMOOTHING / (VOCAB - 1)
C0 = ((N_TOTAL - ROWS) * FILL * math.log(FILL)
      + ROWS * CONFIDENCE * math.log(CONFIDENCE)) / N_TOTAL

VB = 5000
GRID = VOCAB // VB


def _tc_body(tgt_ref, x_ref, out_ref):
    i = pl.program_id(0)

    @pl.when(i == 0)
    def _init():
        out_ref[0, 0] = 0.0
        out_ref[0, 1] = 0.0

    x = x_ref[...]
    viota = lax.broadcasted_iota(jnp.int32, (VB, ROWS), 0)
    tshift = tgt_ref[...] - i * VB
    out_ref[0, 0] += jnp.sum(x)
    out_ref[0, 1] += jnp.sum(jnp.where(viota == tshift, x, 0.0))

    @pl.when(i == pl.num_programs(0) - 1)
    def _finish():
        s_total = out_ref[0, 0]
        g_total = out_ref[0, 1]
        out_ref[0, 0] = (jnp.float32(C0)
                         - (jnp.float32(FILL) * s_total
                            + jnp.float32(CONFIDENCE - FILL) * g_total)
                         * jnp.float32(1.0 / N_TOTAL))


def kernel(pred, target):
    out = pl.pallas_call(
        _tc_body,
        grid=(GRID,),
        in_specs=[
            pl.BlockSpec((1, ROWS), lambda i: (0, 0)),
            pl.BlockSpec((VB, ROWS), lambda i: (i, 0)),
        ],
        out_specs=pl.BlockSpec(memory_space=pltpu.SMEM),
        out_shape=jax.ShapeDtypeStruct((1, 2), jnp.float32),
    )(target.reshape(1, ROWS), pred.T)
    return out[0, 0]
